# baseline (speedup 1.0000x reference)
import functools

import jax
import jax.numpy as jnp
from jax import lax
from jax.experimental import pallas as pl
from jax.experimental.pallas import tpu as pltpu
from jax.experimental.pallas import tpu_sc as plsc

_GLOBAL_AVG = 3.5
_BATCH = 16384


@jax.jit
def kernel(user_id, item_id, user_bias_score, item_bias_score):
    info = plsc.get_sparse_core_info()
    nc, ns, lanes = info.num_cores, info.num_subcores, info.num_lanes
    nw = nc * ns
    b_per_w = _BATCH // nw

    mesh = plsc.VectorSubcoreMesh(core_axis_name="c", subcore_axis_name="s")
    n_chunks = 4
    csz = b_per_w // n_chunks

    @functools.partial(
        pl.kernel,
        out_type=jax.ShapeDtypeStruct((_BATCH,), jnp.float32),
        mesh=mesh,
        scratch_types=[
            pltpu.VMEM((b_per_w,), jnp.int32),
            pltpu.VMEM((b_per_w,), jnp.int32),
            pltpu.VMEM((b_per_w,), jnp.float32),
            pltpu.VMEM((b_per_w,), jnp.float32),
            pltpu.SemaphoreType.DMA,
            pltpu.SemaphoreType.DMA,
            pltpu.SemaphoreType.DMA((4,)),
            pltpu.SemaphoreType.DMA((4,)),
        ],
    )
    def run(uid_hbm, iid_hbm, utab_hbm, itab_hbm, out_hbm,
            uidx_v, iidx_v, uval_v, ival_v, sem_u, sem_i, gsem_u, gsem_i):
        wid = lax.axis_index("s") * nc + lax.axis_index("c")
        base = wid * b_per_w

        cp_u = pltpu.async_copy(uid_hbm.at[pl.ds(base, b_per_w)], uidx_v, sem_u)
        cp_i = pltpu.async_copy(iid_hbm.at[pl.ds(base, b_per_w)], iidx_v, sem_i)
        cp_u.wait()
        cp_i.wait()

        gathers = []
        for c in range(n_chunks):
            s = pl.ds(c * csz, csz)
            gathers.append(
                (pltpu.async_copy(utab_hbm.at[uidx_v.at[s]], uval_v.at[s], gsem_u.at[c]),
                 pltpu.async_copy(itab_hbm.at[iidx_v.at[s]], ival_v.at[s], gsem_i.at[c])))

        stores = []
        for c in range(n_chunks):
            g_u, g_i = gathers[c]
            g_u.wait()
            g_i.wait()
            for j in range(c * csz, (c + 1) * csz, lanes):
                s = pl.ds(j, lanes)
                uval_v[s] = uval_v[s] + ival_v[s] + _GLOBAL_AVG
            s = pl.ds(c * csz, csz)
            stores.append(pltpu.async_copy(
                uval_v.at[s], out_hbm.at[pl.ds(base + c * csz, csz)], sem_i))
        for st in stores:
            st.wait()

    return run(user_id, item_id, user_bias_score, item_bias_score)


# speedup vs baseline: 1.0064x; 1.0064x over previous
import functools

import jax
import jax.numpy as jnp
from jax import lax
from jax.experimental import pallas as pl
from jax.experimental.pallas import tpu as pltpu
from jax.experimental.pallas import tpu_sc as plsc

_GLOBAL_AVG = 3.5
_BATCH = 16384


@jax.jit
def kernel(user_id, item_id, user_bias_score, item_bias_score):
    info = plsc.get_sparse_core_info()
    nc, ns, lanes = info.num_cores, info.num_subcores, info.num_lanes
    nw = nc * ns
    b_per_w = _BATCH // nw

    mesh = plsc.VectorSubcoreMesh(core_axis_name="c", subcore_axis_name="s")
    n_chunks = 2
    csz = b_per_w // n_chunks

    @functools.partial(
        pl.kernel,
        out_type=jax.ShapeDtypeStruct((_BATCH,), jnp.float32),
        mesh=mesh,
        scratch_types=[
            pltpu.VMEM((b_per_w,), jnp.int32),
            pltpu.VMEM((b_per_w,), jnp.int32),
            pltpu.VMEM((b_per_w,), jnp.float32),
            pltpu.VMEM((b_per_w,), jnp.float32),
            pltpu.SemaphoreType.DMA,
            pltpu.SemaphoreType.DMA,
            pltpu.SemaphoreType.DMA((2,)),
            pltpu.SemaphoreType.DMA((2,)),
        ],
    )
    def run(uid_hbm, iid_hbm, utab_hbm, itab_hbm, out_hbm,
            uidx_v, iidx_v, uval_v, ival_v, sem_u, sem_i, gsem_u, gsem_i):
        wid = lax.axis_index("s") * nc + lax.axis_index("c")
        base = wid * b_per_w

        cp_u = pltpu.async_copy(uid_hbm.at[pl.ds(base, b_per_w)], uidx_v, sem_u)
        cp_i = pltpu.async_copy(iid_hbm.at[pl.ds(base, b_per_w)], iidx_v, sem_i)
        cp_u.wait()
        cp_i.wait()

        gathers = []
        for c in range(n_chunks):
            s = pl.ds(c * csz, csz)
            gathers.append(
                (pltpu.async_copy(utab_hbm.at[uidx_v.at[s]], uval_v.at[s], gsem_u.at[c]),
                 pltpu.async_copy(itab_hbm.at[iidx_v.at[s]], ival_v.at[s], gsem_i.at[c])))

        stores = []
        for c in range(n_chunks):
            g_u, g_i = gathers[c]
            g_u.wait()
            g_i.wait()
            for j in range(c * csz, (c + 1) * csz, lanes):
                s = pl.ds(j, lanes)
                uval_v[s] = uval_v[s] + ival_v[s] + _GLOBAL_AVG
            s = pl.ds(c * csz, csz)
            stores.append(pltpu.async_copy(
                uval_v.at[s], out_hbm.at[pl.ds(base + c * csz, csz)], sem_i))
        for st in stores:
            st.wait()

    return run(user_id, item_id, user_bias_score, item_bias_score)


# speedup vs baseline: 1.0098x; 1.0033x over previous
import functools

import jax
import jax.numpy as jnp
from jax import lax
from jax.experimental import pallas as pl
from jax.experimental.pallas import tpu as pltpu
from jax.experimental.pallas import tpu_sc as plsc

_GLOBAL_AVG = 3.5
_BATCH = 16384


@jax.jit
def kernel(user_id, item_id, user_bias_score, item_bias_score):
    info = plsc.get_sparse_core_info()
    nc, ns, lanes = info.num_cores, info.num_subcores, info.num_lanes
    nw = nc * ns
    b_per_w = _BATCH // nw

    mesh = plsc.VectorSubcoreMesh(core_axis_name="c", subcore_axis_name="s")

    @functools.partial(
        pl.kernel,
        out_type=jax.ShapeDtypeStruct((_BATCH,), jnp.float32),
        mesh=mesh,
        scratch_types=[
            pltpu.VMEM((b_per_w,), jnp.int32),
            pltpu.VMEM((b_per_w,), jnp.int32),
            pltpu.VMEM((b_per_w,), jnp.float32),
            pltpu.VMEM((b_per_w,), jnp.float32),
            pltpu.SemaphoreType.DMA,
            pltpu.SemaphoreType.DMA,
            pltpu.SemaphoreType.DMA,
            pltpu.SemaphoreType.DMA,
            pltpu.SemaphoreType.DMA,
        ],
    )
    def run(uid_hbm, iid_hbm, utab_hbm, itab_hbm, out_hbm,
            uidx_v, iidx_v, uval_v, ival_v, sem_u, sem_i, sem_u2, sem_i2, sem_st):
        wid = lax.axis_index("s") * nc + lax.axis_index("c")
        base = wid * b_per_w
        big = 384

        cp_u = pltpu.async_copy(uid_hbm.at[pl.ds(base, b_per_w)], uidx_v, sem_u)
        cp_i = pltpu.async_copy(iid_hbm.at[pl.ds(base, b_per_w)], iidx_v, sem_i)
        s0 = pl.ds(0, big)
        s1 = pl.ds(big, b_per_w - big)
        cp_u.wait()
        g_u0 = pltpu.async_copy(utab_hbm.at[uidx_v.at[s0]], uval_v.at[s0], sem_u)
        g_u1 = pltpu.async_copy(utab_hbm.at[uidx_v.at[s1]], uval_v.at[s1], sem_u2)
        cp_i.wait()
        g_i0 = pltpu.async_copy(itab_hbm.at[iidx_v.at[s0]], ival_v.at[s0], sem_i)
        g_i1 = pltpu.async_copy(itab_hbm.at[iidx_v.at[s1]], ival_v.at[s1], sem_i2)

        g_u0.wait()
        g_i0.wait()
        for j in range(0, big, lanes):
            s = pl.ds(j, lanes)
            uval_v[s] = uval_v[s] + ival_v[s] + _GLOBAL_AVG
        st0 = pltpu.async_copy(uval_v.at[s0], out_hbm.at[pl.ds(base, big)], sem_st)

        g_u1.wait()
        g_i1.wait()
        for j in range(big, b_per_w, lanes):
            s = pl.ds(j, lanes)
            uval_v[s] = uval_v[s] + ival_v[s] + _GLOBAL_AVG
        pltpu.sync_copy(uval_v.at[s1], out_hbm.at[pl.ds(base + big, b_per_w - big)])
        st0.wait()

    return run(user_id, item_id, user_bias_score, item_bias_score)


# speedup vs baseline: 1.0282x; 1.0182x over previous
import dataclasses

import jax
import jax.numpy as jnp
from jax import lax
from jax.experimental import pallas as pl
from jax.experimental.pallas import tpu as pltpu
from jax.experimental.pallas import tpu_sc as plsc
from jax._src.pallas import core as _pl_core

_GLOBAL_AVG = 3.5
_BATCH = 16384


def _on(mesh, mem_ref):
    return dataclasses.replace(
        mem_ref,
        memory_space=_pl_core.CoreMemorySpace(mem_ref.memory_space, mesh))


@jax.jit
def kernel(user_id, item_id, user_bias_score, item_bias_score):
    info = plsc.get_sparse_core_info()
    nc, ns, lanes = info.num_cores, info.num_subcores, info.num_lanes
    nw = nc * ns
    b_per_w = _BATCH // nw
    half = _BATCH // nc

    vmesh = plsc.VectorSubcoreMesh(core_axis_name="c", subcore_axis_name="s")
    smesh = plsc.ScalarSubcoreMesh(axis_name="c", num_cores=nc)

    def scs_fn(uid_hbm, iid_hbm, utab_hbm, itab_hbm, out_hbm,
               sidx_u, sidx_i, scs_sem_u, scs_sem_i, ready,
               uidx_v, iidx_v, uval_v, ival_v, sem_u, sem_i):
        c = lax.axis_index("c")
        cp_u = pltpu.async_copy(uid_hbm.at[pl.ds(c * half, half)], sidx_u,
                                scs_sem_u)
        cp_i = pltpu.async_copy(iid_hbm.at[pl.ds(c * half, half)], sidx_i,
                                scs_sem_i)
        cp_u.wait()
        cp_i.wait()
        for t in range(ns):
            pl.semaphore_signal(ready, 1, device_id={"s": t})

    def tec_fn(uid_hbm, iid_hbm, utab_hbm, itab_hbm, out_hbm,
               sidx_u, sidx_i, scs_sem_u, scs_sem_i, ready,
               uidx_v, iidx_v, uval_v, ival_v, sem_u, sem_i):
        c = lax.axis_index("c")
        s = lax.axis_index("s")
        base = c * half + s * b_per_w
        loc = pl.ds(s * b_per_w, b_per_w)

        pl.semaphore_wait(ready, 1)
        pltpu.sync_copy(sidx_u.at[loc], uidx_v)
        g_u = pltpu.async_copy(utab_hbm.at[uidx_v], uval_v, sem_u)
        pltpu.sync_copy(sidx_i.at[loc], iidx_v)
        g_i = pltpu.async_copy(itab_hbm.at[iidx_v], ival_v, sem_i)
        g_u.wait()
        g_i.wait()

        @pl.loop(0, b_per_w, step=lanes)
        def _(j):
            sl = pl.ds(j, lanes)
            uval_v[sl] = uval_v[sl] + ival_v[sl] + _GLOBAL_AVG

        pltpu.sync_copy(uval_v, out_hbm.at[pl.ds(base, b_per_w)])

    run = pl.kernel(
        body=[tec_fn, scs_fn],
        mesh=[vmesh, smesh],
        out_type=jax.ShapeDtypeStruct((_BATCH,), jnp.float32),
        scratch_types=[
            pltpu.VMEM_SHARED((half,), jnp.int32),
            pltpu.VMEM_SHARED((half,), jnp.int32),
            _on(smesh, pltpu.SemaphoreType.DMA(())),
            _on(smesh, pltpu.SemaphoreType.DMA(())),
            _on(vmesh, pltpu.SemaphoreType.REGULAR(())),
            _on(vmesh, pltpu.VMEM((b_per_w,), jnp.int32)),
            _on(vmesh, pltpu.VMEM((b_per_w,), jnp.int32)),
            _on(vmesh, pltpu.VMEM((b_per_w,), jnp.float32)),
            _on(vmesh, pltpu.VMEM((b_per_w,), jnp.float32)),
            _on(vmesh, pltpu.SemaphoreType.DMA(())),
            _on(vmesh, pltpu.SemaphoreType.DMA(())),
        ],
    )
    return run(user_id, item_id, user_bias_score, item_bias_score)
